# baseline (speedup 1.0000x reference)
import functools

import jax
import jax.numpy as jnp
from jax import lax
from jax.experimental import pallas as pl
from jax.experimental.pallas import tpu as pltpu
from jax.experimental.pallas import tpu_sc as plsc

N = 10000
E = 320000
IN_DIM = 128
HID_DIM = 128
OUT_DIM = 64

NC = 2
NS = 16
NW = NC * NS
CH = 128
ROWS_PER_SUB = 632
NPAD = NS * ROWS_PER_SUB
STEPS = 80
EPW = STEPS * CH
EPAD = EPW * NW
IBLK = 8
NIB = STEPS // IBLK
C0_STEPS = 160
C1_STEPS = 0


def _vmesh():
    return plsc.VectorSubcoreMesh(core_axis_name="c", subcore_axis_name="s")


def _make_edge_agg(D):
    @functools.partial(
        pl.kernel,
        out_type=jax.ShapeDtypeStruct((NC, NPAD, D), jnp.float32),
        mesh=_vmesh(),
        scratch_types=[
            pltpu.VMEM((IBLK, CH), jnp.int32),
            pltpu.VMEM((IBLK, CH), jnp.int32),
            pltpu.VMEM((IBLK, CH), jnp.int32),
            pltpu.VMEM((IBLK, CH), jnp.int32),
            pltpu.VMEM((CH, D), jnp.float32),
            pltpu.VMEM((CH, D), jnp.float32),
            pltpu.VMEM_SHARED((NPAD, D), jnp.float32),
            pltpu.SemaphoreType.DMA,
            pltpu.SemaphoreType.DMA,
            pltpu.SemaphoreType.DMA,
            pltpu.SemaphoreType.DMA,
        ],
    )
    def agg(h_hbm, src_hbm, dst_hbm, zer_hbm, out_hbm,
            ss0, ss1, ds0, ds1, buf0, buf1, acc_sh,
            isem0, isem1, gsem0, gsem1):
        sslot = (ss0, ss1)
        dslot = (ds0, ds1)
        bufs = (buf0, buf1)
        isems = (isem0, isem1)
        gsems = (gsem0, gsem1)
        cid = lax.axis_index("c")
        sid = lax.axis_index("s")
        row0 = sid * ROWS_PER_SUB
        is0 = cid == 0
        tbase = jnp.where(
            is0, sid * C0_STEPS,
            jnp.minimum(NS * C0_STEPS + sid * C1_STEPS,
                        EPAD // CH - 2 * IBLK))
        nib2 = jnp.where(is0, C0_STEPS // (2 * IBLK), C1_STEPS // (2 * IBLK))
        nib = nib2 * 2

        def _idx_load(j, slot):
            r = pl.multiple_of(tbase + j * IBLK, IBLK)
            pltpu.async_copy(src_hbm.at[pl.ds(r, IBLK)], sslot[slot], isems[slot])
            pltpu.async_copy(dst_hbm.at[pl.ds(r, IBLK)], dslot[slot], isems[slot])

        def _idx_wait(slot):
            pltpu.make_async_copy(src_hbm.at[pl.ds(0, IBLK)], sslot[slot],
                                  isems[slot]).wait()
            pltpu.make_async_copy(dst_hbm.at[pl.ds(0, IBLK)], dslot[slot],
                                  isems[slot]).wait()

        SP = 4
        SW = CH // SP

        def _gather(slot, b, k):
            for h in range(SP):
                pltpu.async_copy(
                    h_hbm.at[sslot[slot].at[k, pl.ds(h * SW, SW)]],
                    bufs[b].at[pl.ds(h * SW, SW)], gsems[b])

        def _gather_wait(b):
            for h in range(SP):
                pltpu.make_async_copy(h_hbm.at[pl.ds(0, SW)],
                                      bufs[b].at[pl.ds(h * SW, SW)],
                                      gsems[b]).wait()

        _idx_load(0, 0)
        _idx_load(1, 1)
        pltpu.sync_copy(zer_hbm, acc_sh.at[pl.ds(row0, ROWS_PER_SUB)])
        plsc.subcore_barrier()

        @pl.loop(0, nib2)
        def _(i):
            for slot in range(2):
                j = 2 * i + slot
                _idx_wait(slot)
                _gather(slot, 0, 0)
                _gather(slot, 1, 1)

                @pl.loop(0, IBLK // 2)
                def _(m):
                    k = 2 * m
                    for b in range(2):
                        _gather_wait(b)
                        pltpu.sync_copy(bufs[b],
                                        acc_sh.at[dslot[slot].at[k + b]],
                                        add=True)

                        @pl.when(k + 2 + b < IBLK)
                        def _():
                            _gather(slot, b, k + 2 + b)

                @pl.when(j + 2 < nib)
                def _():
                    _idx_load(j + 2, slot)

        plsc.subcore_barrier()
        pltpu.sync_copy(acc_sh.at[pl.ds(row0, ROWS_PER_SUB)],
                        out_hbm.at[cid].at[pl.ds(row0, ROWS_PER_SUB)])

    return agg


_agg_hid = _make_edge_agg(HID_DIM)


@functools.partial(
    pl.kernel,
    out_type=jax.ShapeDtypeStruct((NC, NPAD, HID_DIM), jnp.float32),
    mesh=_vmesh(),
    scratch_types=[
        pltpu.VMEM((STEPS, CH), jnp.int32),
        pltpu.VMEM((CH, HID_DIM), jnp.float32),
        pltpu.VMEM_SHARED((NPAD, HID_DIM), jnp.float32),
    ],
)
def _deg_kernel(dst_hbm, ones_hbm, zer_hbm, out_hbm, dst_t, ones_v, acc_sh):
    cid = lax.axis_index("c")
    sid = lax.axis_index("s")
    wid = sid * NC + cid
    row0 = sid * ROWS_PER_SUB
    pltpu.sync_copy(dst_hbm.at[pl.ds(wid * STEPS, STEPS)], dst_t)
    pltpu.sync_copy(ones_hbm, ones_v)
    pltpu.sync_copy(zer_hbm, acc_sh.at[pl.ds(row0, ROWS_PER_SUB)])
    plsc.subcore_barrier()

    @pl.loop(0, STEPS)
    def _(step):
        pltpu.sync_copy(ones_v, acc_sh.at[dst_t.at[step]], add=True)

    plsc.subcore_barrier()
    pltpu.sync_copy(acc_sh.at[pl.ds(row0, ROWS_PER_SUB)],
                    out_hbm.at[cid].at[pl.ds(row0, ROWS_PER_SUB)])


def _dot(a, b):
    return lax.dot_general(a, b, (((1,), (0,)), ((), ())),
                           precision=lax.Precision.HIGHEST,
                           preferred_element_type=jnp.float32)


def _mm1_body(x_ref, w_ref, o_ref):
    o_ref[...] = _dot(x_ref[...], w_ref[...])


_mm1 = pl.pallas_call(
    _mm1_body,
    out_shape=jax.ShapeDtypeStruct((NPAD, HID_DIM), jnp.float32),
)


def _prep_body(h_ref, degp_ref, dinv_ref, h1p_ref):
    deg = degp_ref[0, :, 0:1] + degp_ref[1, :, 0:1] + 1.0
    dinv = jnp.broadcast_to(lax.rsqrt(deg), (NPAD, HID_DIM))
    dinv_ref[...] = dinv
    h1p_ref[...] = h_ref[...] * dinv


_prep = pl.pallas_call(
    _prep_body,
    out_shape=(
        jax.ShapeDtypeStruct((NPAD, HID_DIM), jnp.float32),
        jax.ShapeDtypeStruct((NPAD, HID_DIM), jnp.float32),
    ),
)


def _mid_body(acc_ref, h1p_ref, dinv_ref, b1_ref, o_ref):
    z = (acc_ref[0] + acc_ref[1] + h1p_ref[...]) * dinv_ref[...] + b1_ref[...]
    o_ref[...] = jnp.maximum(z, 0.0) * dinv_ref[...]


_mid = pl.pallas_call(
    _mid_body,
    out_shape=jax.ShapeDtypeStruct((NPAD, HID_DIM), jnp.float32),
)


def _out_body(acc_ref, u_ref, dinv_ref, b2_ref, w2_ref, o_ref):
    v = (acc_ref[0] + acc_ref[1] + u_ref[...]) * dinv_ref[...]
    o_ref[...] = (_dot(v, w2_ref[...]) + b2_ref[...])[:N]


_outk = pl.pallas_call(
    _out_body,
    out_shape=jax.ShapeDtypeStruct((N, OUT_DIM), jnp.float32),
)


def kernel(x, edge_index, W1, b1, W2, b2):
    src = edge_index[0].astype(jnp.int32)
    dst = edge_index[1].astype(jnp.int32)
    pad = jnp.full((EPAD - E,), N, jnp.int32)
    src_p = jnp.concatenate([src, pad]).reshape(NW * STEPS, CH)
    dst_p = jnp.concatenate([dst, pad]).reshape(NW * STEPS, CH)
    x_pad = jnp.pad(x, ((0, NPAD - N), (0, 0)))

    zer_h = jnp.zeros((ROWS_PER_SUB, HID_DIM), jnp.float32)
    ones_d = jnp.ones((CH, HID_DIM), jnp.float32)

    degp = _deg_kernel(dst_p, ones_d, zer_h)
    h1 = _mm1(x_pad, W1)
    dinv, h1p = _prep(h1, degp)
    acc1 = _agg_hid(h1p, src_p, dst_p, zer_h)
    u = _mid(acc1, h1p, dinv, b1.reshape(1, HID_DIM))
    acc2 = _agg_hid(u, src_p, dst_p, zer_h)
    return _outk(acc2, u, dinv, b2.reshape(1, OUT_DIM), W2)


# speedup vs baseline: 1.2048x; 1.2048x over previous
import functools

import jax
import jax.numpy as jnp
from jax import lax
from jax.experimental import pallas as pl
from jax.experimental.pallas import tpu as pltpu
from jax.experimental.pallas import tpu_sc as plsc

N = 10000
E = 320000
IN_DIM = 128
HID_DIM = 128
OUT_DIM = 64

NC = 2
NS = 16
NW = NC * NS
CH = 128
ROWS_PER_SUB = 632
NPAD = NS * ROWS_PER_SUB
STEPS = 80
EPW = STEPS * CH
EPAD = EPW * NW
IBLK = 8
NIB = STEPS // IBLK
C0_STEPS = 128
C1_STEPS = 32
C0_NIB = C0_STEPS // IBLK
C1_NIB = C1_STEPS // IBLK


def _vmesh():
    return plsc.VectorSubcoreMesh(core_axis_name="c", subcore_axis_name="s")


def _make_edge_agg(D):
    @functools.partial(
        pl.kernel,
        out_type=jax.ShapeDtypeStruct((NC, NPAD, D), jnp.float32),
        mesh=_vmesh(),
        scratch_types=[
            pltpu.VMEM((IBLK, CH), jnp.int32),
            pltpu.VMEM((IBLK, CH), jnp.int32),
            pltpu.VMEM((IBLK, CH), jnp.int32),
            pltpu.VMEM((IBLK, CH), jnp.int32),
            pltpu.VMEM((CH, D), jnp.float32),
            pltpu.VMEM((CH, D), jnp.float32),
            pltpu.VMEM_SHARED((NPAD, D), jnp.float32),
            pltpu.SemaphoreType.DMA,
            pltpu.SemaphoreType.DMA,
            pltpu.SemaphoreType.DMA,
            pltpu.SemaphoreType.DMA,
        ],
    )
    def agg(h_hbm, src_hbm, dst_hbm, zer_hbm, out_hbm,
            ss0, ss1, ds0, ds1, buf0, buf1, acc_sh,
            isem0, isem1, gsem0, gsem1):
        sslot = (ss0, ss1)
        dslot = (ds0, ds1)
        bufs = (buf0, buf1)
        isems = (isem0, isem1)
        gsems = (gsem0, gsem1)
        cid = lax.axis_index("c")
        sid = lax.axis_index("s")
        row0 = sid * ROWS_PER_SUB
        is0 = cid == 0

        def _idx_load(tbase, j, slot):
            r = pl.multiple_of(tbase + j * IBLK, IBLK)
            pltpu.async_copy(src_hbm.at[pl.ds(r, IBLK)], sslot[slot], isems[slot])
            pltpu.async_copy(dst_hbm.at[pl.ds(r, IBLK)], dslot[slot], isems[slot])

        def _idx_wait(slot):
            pltpu.make_async_copy(src_hbm.at[pl.ds(0, IBLK)], sslot[slot],
                                  isems[slot]).wait()
            pltpu.make_async_copy(dst_hbm.at[pl.ds(0, IBLK)], dslot[slot],
                                  isems[slot]).wait()

        SP = 4
        SW = CH // SP

        def _gather(slot, b, k):
            for h in range(SP):
                pltpu.async_copy(
                    h_hbm.at[sslot[slot].at[k, pl.ds(h * SW, SW)]],
                    bufs[b].at[pl.ds(h * SW, SW)], gsems[b])

        def _gather_wait(b):
            for h in range(SP):
                pltpu.make_async_copy(h_hbm.at[pl.ds(0, SW)],
                                      bufs[b].at[pl.ds(h * SW, SW)],
                                      gsems[b]).wait()

        pltpu.sync_copy(zer_hbm, acc_sh.at[pl.ds(row0, ROWS_PER_SUB)])
        plsc.subcore_barrier()

        @pl.when(is0)
        def _():
            tb = sid * C0_STEPS
            _idx_load(tb, 0, 0)
            _idx_load(tb, 1, 1)

            @pl.loop(0, C0_NIB // 2)
            def _(i):
                for slot in range(2):
                    j = 2 * i + slot
                    _idx_wait(slot)
                    _gather(slot, 0, 0)
                    _gather(slot, 1, 1)

                    @pl.loop(0, IBLK // 2)
                    def _(m):
                        k = 2 * m
                        for b in range(2):
                            _gather_wait(b)
                            pltpu.sync_copy(bufs[b],
                                            acc_sh.at[dslot[slot].at[k + b]],
                                            add=True)

                            @pl.when(k + 2 + b < IBLK)
                            def _():
                                _gather(slot, b, k + 2 + b)

                    @pl.when(j + 2 < C0_NIB)
                    def _():
                        _idx_load(tb, j + 2, slot)

        @pl.when(jnp.logical_not(is0))
        def _():
            tb = NS * C0_STEPS + sid * C1_STEPS

            @pl.loop(0, C1_NIB)
            def _(j):
                r = pl.multiple_of(tb + j * IBLK, IBLK)
                pltpu.sync_copy(src_hbm.at[pl.ds(r, IBLK)], ss0)
                pltpu.sync_copy(dst_hbm.at[pl.ds(r, IBLK)], ds0)

                @pl.loop(0, IBLK)
                def _(k):
                    pltpu.async_copy(h_hbm.at[ss0.at[k]], buf0, gsem0)
                    pltpu.make_async_copy(h_hbm.at[pl.ds(0, CH)], buf0,
                                          gsem0).wait()
                    pltpu.sync_copy(buf0, acc_sh.at[ds0.at[k]], add=True)

        plsc.subcore_barrier()
        pltpu.sync_copy(acc_sh.at[pl.ds(row0, ROWS_PER_SUB)],
                        out_hbm.at[cid].at[pl.ds(row0, ROWS_PER_SUB)])

    return agg


_agg_hid = _make_edge_agg(HID_DIM)


@functools.partial(
    pl.kernel,
    out_type=jax.ShapeDtypeStruct((NC, NPAD, HID_DIM), jnp.float32),
    mesh=_vmesh(),
    scratch_types=[
        pltpu.VMEM((STEPS, CH), jnp.int32),
        pltpu.VMEM((CH, HID_DIM), jnp.float32),
        pltpu.VMEM_SHARED((NPAD, HID_DIM), jnp.float32),
    ],
)
def _deg_kernel(dst_hbm, ones_hbm, zer_hbm, out_hbm, dst_t, ones_v, acc_sh):
    cid = lax.axis_index("c")
    sid = lax.axis_index("s")
    wid = sid * NC + cid
    row0 = sid * ROWS_PER_SUB
    pltpu.sync_copy(dst_hbm.at[pl.ds(wid * STEPS, STEPS)], dst_t)
    pltpu.sync_copy(ones_hbm, ones_v)
    pltpu.sync_copy(zer_hbm, acc_sh.at[pl.ds(row0, ROWS_PER_SUB)])
    plsc.subcore_barrier()

    @pl.loop(0, STEPS)
    def _(step):
        pltpu.sync_copy(ones_v, acc_sh.at[dst_t.at[step]], add=True)

    plsc.subcore_barrier()
    pltpu.sync_copy(acc_sh.at[pl.ds(row0, ROWS_PER_SUB)],
                    out_hbm.at[cid].at[pl.ds(row0, ROWS_PER_SUB)])


def _dot(a, b):
    return lax.dot_general(a, b, (((1,), (0,)), ((), ())),
                           precision=lax.Precision.HIGHEST,
                           preferred_element_type=jnp.float32)


def _mm1_body(x_ref, w_ref, o_ref):
    o_ref[...] = _dot(x_ref[...], w_ref[...])


_mm1 = pl.pallas_call(
    _mm1_body,
    out_shape=jax.ShapeDtypeStruct((NPAD, HID_DIM), jnp.float32),
)


def _prep_body(h_ref, degp_ref, dinv_ref, h1p_ref):
    deg = degp_ref[0, :, 0:1] + degp_ref[1, :, 0:1] + 1.0
    dinv = jnp.broadcast_to(lax.rsqrt(deg), (NPAD, HID_DIM))
    dinv_ref[...] = dinv
    h1p_ref[...] = h_ref[...] * dinv


_prep = pl.pallas_call(
    _prep_body,
    out_shape=(
        jax.ShapeDtypeStruct((NPAD, HID_DIM), jnp.float32),
        jax.ShapeDtypeStruct((NPAD, HID_DIM), jnp.float32),
    ),
)


def _mid_body(acc_ref, h1p_ref, dinv_ref, b1_ref, o_ref):
    z = (acc_ref[0] + acc_ref[1] + h1p_ref[...]) * dinv_ref[...] + b1_ref[...]
    o_ref[...] = jnp.maximum(z, 0.0) * dinv_ref[...]


_mid = pl.pallas_call(
    _mid_body,
    out_shape=jax.ShapeDtypeStruct((NPAD, HID_DIM), jnp.float32),
)


def _out_body(acc_ref, u_ref, dinv_ref, b2_ref, w2_ref, o_ref):
    v = (acc_ref[0] + acc_ref[1] + u_ref[...]) * dinv_ref[...]
    o_ref[...] = (_dot(v, w2_ref[...]) + b2_ref[...])[:N]


_outk = pl.pallas_call(
    _out_body,
    out_shape=jax.ShapeDtypeStruct((N, OUT_DIM), jnp.float32),
)


def kernel(x, edge_index, W1, b1, W2, b2):
    src = edge_index[0].astype(jnp.int32)
    dst = edge_index[1].astype(jnp.int32)
    pad = jnp.full((EPAD - E,), N, jnp.int32)
    src_p = jnp.concatenate([src, pad]).reshape(NW * STEPS, CH)
    dst_p = jnp.concatenate([dst, pad]).reshape(NW * STEPS, CH)
    x_pad = jnp.pad(x, ((0, NPAD - N), (0, 0)))

    zer_h = jnp.zeros((ROWS_PER_SUB, HID_DIM), jnp.float32)
    ones_d = jnp.ones((CH, HID_DIM), jnp.float32)

    degp = _deg_kernel(dst_p, ones_d, zer_h)
    h1 = _mm1(x_pad, W1)
    dinv, h1p = _prep(h1, degp)
    acc1 = _agg_hid(h1p, src_p, dst_p, zer_h)
    u = _mid(acc1, h1p, dinv, b1.reshape(1, HID_DIM))
    acc2 = _agg_hid(u, src_p, dst_p, zer_h)
    return _outk(acc2, u, dinv, b2.reshape(1, OUT_DIM), W2)


# speedup vs baseline: 2.9417x; 2.4417x over previous
import functools

import jax
import jax.numpy as jnp
from jax import lax
from jax.experimental import pallas as pl
from jax.experimental.pallas import tpu as pltpu
from jax.experimental.pallas import tpu_sc as plsc

N = 10000
E = 320000
IN_DIM = 128
HID_DIM = 128
OUT_DIM = 64

NC = 2
NS = 16
NW = NC * NS
CH = 128
ROWS_PER_SUB = 632
NPAD = NS * ROWS_PER_SUB
STEPS = 80
EPW = STEPS * CH
EPAD = EPW * NW
IBLK = 8
NIB = STEPS // IBLK


def _vmesh():
    return plsc.VectorSubcoreMesh(core_axis_name="c", subcore_axis_name="s")


def _make_edge_agg(D):
    @functools.partial(
        pl.kernel,
        out_type=jax.ShapeDtypeStruct((NC, NPAD, D), jnp.float32),
        mesh=_vmesh(),
        scratch_types=[
            pltpu.VMEM((IBLK, CH), jnp.int32),
            pltpu.VMEM((IBLK, CH), jnp.int32),
            pltpu.VMEM((IBLK, CH), jnp.int32),
            pltpu.VMEM((IBLK, CH), jnp.int32),
            pltpu.VMEM((CH, D), jnp.float32),
            pltpu.VMEM((CH, D), jnp.float32),
            pltpu.VMEM_SHARED((NPAD, D), jnp.float32),
            pltpu.SemaphoreType.DMA,
            pltpu.SemaphoreType.DMA,
            pltpu.SemaphoreType.DMA,
            pltpu.SemaphoreType.DMA,
        ],
    )
    def agg(h_hbm, src_hbm, dst_hbm, zer_hbm, out_hbm,
            ss0, ss1, ds0, ds1, buf0, buf1, acc_sh,
            isem0, isem1, gsem0, gsem1):
        sslot = (ss0, ss1)
        dslot = (ds0, ds1)
        bufs = (buf0, buf1)
        isems = (isem0, isem1)
        gsems = (gsem0, gsem1)
        cid = lax.axis_index("c")
        sid = lax.axis_index("s")
        row0 = sid * ROWS_PER_SUB
        is0 = cid == 0

        def _idx_load(tbase, j, slot):
            r = pl.multiple_of(tbase + j * IBLK, IBLK)
            pltpu.async_copy(src_hbm.at[pl.ds(r, IBLK)], sslot[slot], isems[slot])
            pltpu.async_copy(dst_hbm.at[pl.ds(r, IBLK)], dslot[slot], isems[slot])

        def _idx_wait(slot):
            pltpu.make_async_copy(src_hbm.at[pl.ds(0, IBLK)], sslot[slot],
                                  isems[slot]).wait()
            pltpu.make_async_copy(dst_hbm.at[pl.ds(0, IBLK)], dslot[slot],
                                  isems[slot]).wait()

        SP = 4
        SW = CH // SP

        def _gather(slot, b, k):
            for h in range(SP):
                pltpu.async_copy(
                    h_hbm.at[sslot[slot].at[k, pl.ds(h * SW, SW)]],
                    bufs[b].at[pl.ds(h * SW, SW)], gsems[b])

        def _gather_wait(b):
            for h in range(SP):
                pltpu.make_async_copy(h_hbm.at[pl.ds(0, SW)],
                                      bufs[b].at[pl.ds(h * SW, SW)],
                                      gsems[b]).wait()

        pltpu.sync_copy(zer_hbm, acc_sh.at[pl.ds(row0, ROWS_PER_SUB)])
        tb = (sid * NC + cid) * STEPS
        _idx_load(tb, 0, 0)
        _idx_load(tb, 1, 1)
        plsc.subcore_barrier()

        @pl.loop(0, NIB // 2)
        def _(i):
            for slot in range(2):
                j = 2 * i + slot
                _idx_wait(slot)
                _gather(slot, 0, 0)
                _gather(slot, 1, 1)

                @pl.loop(0, IBLK // 2)
                def _(m):
                    k = 2 * m
                    for b in range(2):
                        _gather_wait(b)
                        pltpu.sync_copy(bufs[b],
                                        acc_sh.at[dslot[slot].at[k + b]],
                                        add=True)

                        @pl.when(k + 2 + b < IBLK)
                        def _():
                            _gather(slot, b, k + 2 + b)

                @pl.when(j + 2 < NIB)
                def _():
                    _idx_load(tb, j + 2, slot)

        plsc.subcore_barrier()
        pltpu.sync_copy(acc_sh.at[pl.ds(row0, ROWS_PER_SUB)],
                        out_hbm.at[cid].at[pl.ds(row0, ROWS_PER_SUB)])

    return agg


_agg_hid = _make_edge_agg(HID_DIM)


@functools.partial(
    pl.kernel,
    out_type=jax.ShapeDtypeStruct((NC, NPAD, HID_DIM), jnp.float32),
    mesh=_vmesh(),
    scratch_types=[
        pltpu.VMEM((STEPS, CH), jnp.int32),
        pltpu.VMEM((CH, HID_DIM), jnp.float32),
        pltpu.VMEM_SHARED((NPAD, HID_DIM), jnp.float32),
    ],
)
def _deg_kernel(dst_hbm, ones_hbm, zer_hbm, out_hbm, dst_t, ones_v, acc_sh):
    cid = lax.axis_index("c")
    sid = lax.axis_index("s")
    wid = sid * NC + cid
    row0 = sid * ROWS_PER_SUB
    pltpu.sync_copy(dst_hbm.at[pl.ds(wid * STEPS, STEPS)], dst_t)
    pltpu.sync_copy(ones_hbm, ones_v)
    pltpu.sync_copy(zer_hbm, acc_sh.at[pl.ds(row0, ROWS_PER_SUB)])
    plsc.subcore_barrier()

    @pl.loop(0, STEPS)
    def _(step):
        pltpu.sync_copy(ones_v, acc_sh.at[dst_t.at[step]], add=True)

    plsc.subcore_barrier()
    pltpu.sync_copy(acc_sh.at[pl.ds(row0, ROWS_PER_SUB)],
                    out_hbm.at[cid].at[pl.ds(row0, ROWS_PER_SUB)])


def _dot(a, b):
    return lax.dot_general(a, b, (((1,), (0,)), ((), ())),
                           precision=lax.Precision.HIGHEST,
                           preferred_element_type=jnp.float32)


def _mm1_body(x_ref, w_ref, o_ref):
    o_ref[...] = _dot(x_ref[...], w_ref[...])


_mm1 = pl.pallas_call(
    _mm1_body,
    out_shape=jax.ShapeDtypeStruct((NPAD, HID_DIM), jnp.float32),
)


def _prep_body(h_ref, degp_ref, dinv_ref, h1p_ref):
    deg = degp_ref[0, :, 0:1] + degp_ref[1, :, 0:1] + 1.0
    dinv = jnp.broadcast_to(lax.rsqrt(deg), (NPAD, HID_DIM))
    dinv_ref[...] = dinv
    h1p_ref[...] = h_ref[...] * dinv


_prep = pl.pallas_call(
    _prep_body,
    out_shape=(
        jax.ShapeDtypeStruct((NPAD, HID_DIM), jnp.float32),
        jax.ShapeDtypeStruct((NPAD, HID_DIM), jnp.float32),
    ),
)


def _mid_body(acc_ref, h1p_ref, dinv_ref, b1_ref, o_ref):
    z = (acc_ref[0] + acc_ref[1] + h1p_ref[...]) * dinv_ref[...] + b1_ref[...]
    rows = lax.broadcasted_iota(jnp.int32, (NPAD, HID_DIM), 0)
    o_ref[...] = jnp.where(rows < N, jnp.maximum(z, 0.0) * dinv_ref[...], 0.0)


_mid = pl.pallas_call(
    _mid_body,
    out_shape=jax.ShapeDtypeStruct((NPAD, HID_DIM), jnp.float32),
)


def _out_body(acc_ref, u_ref, dinv_ref, b2_ref, w2_ref, o_ref):
    v = (acc_ref[0] + acc_ref[1] + u_ref[...]) * dinv_ref[...]
    o_ref[...] = (_dot(v, w2_ref[...]) + b2_ref[...])[:N]


_outk = pl.pallas_call(
    _out_body,
    out_shape=jax.ShapeDtypeStruct((N, OUT_DIM), jnp.float32),
)


def kernel(x, edge_index, W1, b1, W2, b2):
    src = edge_index[0].astype(jnp.int32)
    dst = edge_index[1].astype(jnp.int32)
    ar = jnp.arange(EPAD - E, dtype=jnp.int32)
    pad_src = N + ar % (NPAD - N)
    pad_dst = ar % N
    src_p = jnp.concatenate([src, pad_src]).reshape(NW * STEPS, CH)
    dst_p = jnp.concatenate([dst, pad_dst]).reshape(NW * STEPS, CH)
    dst_deg = jnp.concatenate(
        [dst, jnp.full((EPAD - E,), N, jnp.int32)]).reshape(NW * STEPS, CH)
    x_pad = jnp.pad(x, ((0, NPAD - N), (0, 0)))

    zer_h = jnp.zeros((ROWS_PER_SUB, HID_DIM), jnp.float32)
    ones_d = jnp.ones((CH, HID_DIM), jnp.float32)

    degp = _deg_kernel(dst_deg, ones_d, zer_h)
    h1 = _mm1(x_pad, W1)
    dinv, h1p = _prep(h1, degp)
    acc1 = _agg_hid(h1p, src_p, dst_p, zer_h)
    u = _mid(acc1, h1p, dinv, b1.reshape(1, HID_DIM))
    acc2 = _agg_hid(u, src_p, dst_p, zer_h)
    return _outk(acc2, u, dinv, b2.reshape(1, OUT_DIM), W2)


# speedup vs baseline: 2.9458x; 1.0014x over previous
import functools

import jax
import jax.numpy as jnp
from jax import lax
from jax.experimental import pallas as pl
from jax.experimental.pallas import tpu as pltpu
from jax.experimental.pallas import tpu_sc as plsc

N = 10000
E = 320000
IN_DIM = 128
HID_DIM = 128
OUT_DIM = 64

NC = 2
NS = 16
NW = NC * NS
CH = 128
ROWS_PER_SUB = 632
NPAD = NS * ROWS_PER_SUB
STEPS = 80
EPW = STEPS * CH
EPAD = EPW * NW
IBLK = 8
NIB = STEPS // IBLK


def _vmesh():
    return plsc.VectorSubcoreMesh(core_axis_name="c", subcore_axis_name="s")


def _make_edge_agg(D):
    @functools.partial(
        pl.kernel,
        out_type=jax.ShapeDtypeStruct((NC, NPAD, D), jnp.float32),
        mesh=_vmesh(),
        scratch_types=[
            pltpu.VMEM((IBLK, CH), jnp.int32),
            pltpu.VMEM((IBLK, CH), jnp.int32),
            pltpu.VMEM((IBLK, CH), jnp.int32),
            pltpu.VMEM((IBLK, CH), jnp.int32),
            pltpu.VMEM((CH, D), jnp.float32),
            pltpu.VMEM((CH, D), jnp.float32),
            pltpu.VMEM_SHARED((NPAD, D), jnp.float32),
            pltpu.SemaphoreType.DMA,
            pltpu.SemaphoreType.DMA,
            pltpu.SemaphoreType.DMA,
            pltpu.SemaphoreType.DMA,
        ],
    )
    def agg(h_hbm, src_hbm, dst_hbm, zer_hbm, out_hbm,
            ss0, ss1, ds0, ds1, buf0, buf1, acc_sh,
            isem0, isem1, gsem0, gsem1):
        sslot = (ss0, ss1)
        dslot = (ds0, ds1)
        bufs = (buf0, buf1)
        isems = (isem0, isem1)
        gsems = (gsem0, gsem1)
        cid = lax.axis_index("c")
        sid = lax.axis_index("s")
        row0 = sid * ROWS_PER_SUB

        def _idx_load(tbase, j, slot):
            r = pl.multiple_of(tbase + j * IBLK, IBLK)
            pltpu.async_copy(src_hbm.at[pl.ds(r, IBLK)], sslot[slot], isems[slot])
            pltpu.async_copy(dst_hbm.at[pl.ds(r, IBLK)], dslot[slot], isems[slot])

        def _idx_wait(slot):
            pltpu.make_async_copy(src_hbm.at[pl.ds(0, IBLK)], sslot[slot],
                                  isems[slot]).wait()
            pltpu.make_async_copy(dst_hbm.at[pl.ds(0, IBLK)], dslot[slot],
                                  isems[slot]).wait()

        SP = 4
        SW = CH // SP

        def _gather(slot, b, k):
            for h in range(SP):
                pltpu.async_copy(
                    h_hbm.at[sslot[slot].at[k, pl.ds(h * SW, SW)]],
                    bufs[b].at[pl.ds(h * SW, SW)], gsems[b])

        def _gather_wait(b):
            for h in range(SP):
                pltpu.make_async_copy(h_hbm.at[pl.ds(0, SW)],
                                      bufs[b].at[pl.ds(h * SW, SW)],
                                      gsems[b]).wait()

        pltpu.sync_copy(zer_hbm, acc_sh.at[pl.ds(row0, ROWS_PER_SUB)])
        tb = (sid * NC + cid) * STEPS
        _idx_load(tb, 0, 0)
        _idx_load(tb, 1, 1)
        plsc.subcore_barrier()

        @pl.loop(0, NIB // 2)
        def _(i):
            for slot in range(2):
                j = 2 * i + slot
                _idx_wait(slot)
                _gather(slot, 0, 0)
                _gather(slot, 1, 1)

                @pl.loop(0, IBLK // 2)
                def _(m):
                    k = 2 * m
                    for b in range(2):
                        _gather_wait(b)
                        pltpu.sync_copy(bufs[b],
                                        acc_sh.at[dslot[slot].at[k + b]],
                                        add=True)

                        @pl.when(k + 2 + b < IBLK)
                        def _():
                            _gather(slot, b, k + 2 + b)

                @pl.when(j + 2 < NIB)
                def _():
                    _idx_load(tb, j + 2, slot)

        plsc.subcore_barrier()
        pltpu.sync_copy(acc_sh.at[pl.ds(row0, ROWS_PER_SUB)],
                        out_hbm.at[cid].at[pl.ds(row0, ROWS_PER_SUB)])

    return agg


_agg_hid = _make_edge_agg(HID_DIM)


@functools.partial(
    pl.kernel,
    out_type=jax.ShapeDtypeStruct((NC, NPAD, HID_DIM), jnp.float32),
    mesh=_vmesh(),
    scratch_types=[
        pltpu.VMEM((STEPS, CH), jnp.int32),
        pltpu.VMEM((CH, HID_DIM), jnp.float32),
        pltpu.VMEM_SHARED((NPAD, HID_DIM), jnp.float32),
    ],
)
def _deg_kernel(dst_hbm, ones_hbm, zer_hbm, out_hbm, dst_t, ones_v, acc_sh):
    cid = lax.axis_index("c")
    sid = lax.axis_index("s")
    wid = sid * NC + cid
    row0 = sid * ROWS_PER_SUB
    pltpu.sync_copy(dst_hbm.at[pl.ds(wid * STEPS, STEPS)], dst_t)
    pltpu.sync_copy(ones_hbm, ones_v)
    pltpu.sync_copy(zer_hbm, acc_sh.at[pl.ds(row0, ROWS_PER_SUB)])
    plsc.subcore_barrier()

    @pl.loop(0, STEPS)
    def _(step):
        pltpu.sync_copy(ones_v, acc_sh.at[dst_t.at[step]], add=True)

    plsc.subcore_barrier()
    pltpu.sync_copy(acc_sh.at[pl.ds(row0, ROWS_PER_SUB)],
                    out_hbm.at[cid].at[pl.ds(row0, ROWS_PER_SUB)])


def _dot(a, b):
    return lax.dot_general(a, b, (((1,), (0,)), ((), ())),
                           precision=lax.Precision.HIGHEST,
                           preferred_element_type=jnp.float32)


def _mm1_body(x_ref, w_ref, o_ref):
    o_ref[...] = _dot(x_ref[...], w_ref[...])


_mm1 = pl.pallas_call(
    _mm1_body,
    out_shape=jax.ShapeDtypeStruct((NPAD, HID_DIM), jnp.float32),
)


def _prep_body(h_ref, degp_ref, dinv_ref, h1p_ref):
    deg = degp_ref[0, :, 0:1] + degp_ref[1, :, 0:1] + 1.0
    dinv = jnp.broadcast_to(lax.rsqrt(deg), (NPAD, HID_DIM))
    dinv_ref[...] = dinv
    h1p_ref[...] = h_ref[...] * dinv


_prep = pl.pallas_call(
    _prep_body,
    out_shape=(
        jax.ShapeDtypeStruct((NPAD, HID_DIM), jnp.float32),
        jax.ShapeDtypeStruct((NPAD, HID_DIM), jnp.float32),
    ),
)


def _mid_body(acc_ref, h1p_ref, dinv_ref, b1_ref, o_ref):
    z = (acc_ref[0] + acc_ref[1] + h1p_ref[...]) * dinv_ref[...] + b1_ref[...]
    rows = lax.broadcasted_iota(jnp.int32, (NPAD, HID_DIM), 0)
    o_ref[...] = jnp.where(rows < N, jnp.maximum(z, 0.0) * dinv_ref[...], 0.0)


_mid = pl.pallas_call(
    _mid_body,
    out_shape=jax.ShapeDtypeStruct((NPAD, HID_DIM), jnp.float32),
)


def _out_body(acc_ref, u_ref, dinv_ref, b2_ref, w2_ref, o_ref):
    v = (acc_ref[0] + acc_ref[1] + u_ref[...]) * dinv_ref[...]
    o_ref[...] = (_dot(v, w2_ref[...]) + b2_ref[...])[:N]


_outk = pl.pallas_call(
    _out_body,
    out_shape=jax.ShapeDtypeStruct((N, OUT_DIM), jnp.float32),
)


def kernel(x, edge_index, W1, b1, W2, b2):
    src = edge_index[0].astype(jnp.int32)
    dst = edge_index[1].astype(jnp.int32)
    ar = jnp.arange(EPAD - E, dtype=jnp.int32)
    pad_src = N + ar % (NPAD - N)
    pad_dst = ar % N
    src_p = jnp.concatenate([src, pad_src]).reshape(NW * STEPS, CH)
    dst_p = jnp.concatenate([dst, pad_dst]).reshape(NW * STEPS, CH)
    dst_deg = jnp.concatenate(
        [dst, jnp.full((EPAD - E,), N, jnp.int32)]).reshape(NW * STEPS, CH)
    x_pad = jnp.pad(x, ((0, NPAD - N), (0, 0)))

    zer_h = jnp.zeros((ROWS_PER_SUB, HID_DIM), jnp.float32)
    ones_d = jnp.ones((CH, HID_DIM), jnp.float32)

    degp = _deg_kernel(dst_deg, ones_d, zer_h)
    h1 = _mm1(x_pad, W1)
    dinv, h1p = _prep(h1, degp)
    acc1 = _agg_hid(h1p, src_p, dst_p, zer_h)
    u = _mid(acc1, h1p, dinv, b1.reshape(1, HID_DIM))
    acc2 = _agg_hid(u, src_p, dst_p, zer_h)
    return _outk(acc2, u, dinv, b2.reshape(1, OUT_DIM), W2)
